# baseline (speedup 1.0000x reference)
import jax
import jax.numpy as jnp
from jax.experimental import pallas as pl
from jax.experimental.pallas import tpu as pltpu

N = 4096
FEATS = 128
HID = 64
LABEL = 10
BLK = 512


def _gcn_kernel(v_ref, adj_ref, w0_ref, wout_ref, bout_ref, out_ref,
                support_ref):
    i = pl.program_id(0)

    @pl.when(i == 0)
    def _init():
        support_ref[:] = jnp.dot(v_ref[:], w0_ref[:],
                                 preferred_element_type=jnp.float32)
        out_ref[:] = bout_ref[:]

    h = jnp.dot(adj_ref[:], support_ref[:],
                preferred_element_type=jnp.float32)
    s = jnp.sum(jnp.maximum(h, 0.0), axis=1)[None, :]
    out_ref[:] += jax.lax.dot_general(
        s, wout_ref[:, pl.ds(i * BLK, BLK)], (((1,), (1,)), ((), ())),
        preferred_element_type=jnp.float32)


def kernel(v, adj, W0, W_out, b_out):
    out = pl.pallas_call(
        _gcn_kernel,
        grid=(N // BLK,),
        in_specs=[
            pl.BlockSpec((N, FEATS), lambda i: (0, 0)),
            pl.BlockSpec((BLK, N), lambda i: (i, 0)),
            pl.BlockSpec((FEATS, HID), lambda i: (0, 0)),
            pl.BlockSpec((LABEL, N), lambda i: (0, 0)),
            pl.BlockSpec((1, LABEL), lambda i: (0, 0)),
        ],
        out_specs=pl.BlockSpec((1, LABEL), lambda i: (0, 0)),
        out_shape=jax.ShapeDtypeStruct((1, LABEL), jnp.float32),
        scratch_shapes=[pltpu.VMEM((N, HID), jnp.float32)],
    )(v, adj, W0, W_out, b_out.reshape(1, LABEL))
    return out.reshape(LABEL)


# speedup vs baseline: 1.0122x; 1.0122x over previous
import jax
import jax.numpy as jnp
from jax.experimental import pallas as pl
from jax.experimental.pallas import tpu as pltpu

N = 4096
FEATS = 128
HID = 64
LABEL = 10
BLK = 512


def _gcn_kernel(v_ref, adj_ref, w0_ref, wout_ref, bout_ref, out_ref,
                support_ref):
    i = pl.program_id(0)

    @pl.when(i == 0)
    def _init():
        support_ref[:] = jnp.dot(v_ref[:], w0_ref[:],
                                 preferred_element_type=jnp.float32)
        out_ref[:] = bout_ref[:]

    h = jnp.dot(adj_ref[:], support_ref[:],
                preferred_element_type=jnp.float32)
    s = jnp.sum(jnp.maximum(h, 0.0), axis=1)[None, :]
    partial = jax.lax.dot_general(
        s, wout_ref[:, pl.ds(i * BLK, BLK)], (((1,), (1,)), ((), ())),
        preferred_element_type=jnp.float32)
    out_ref[:] += partial[0]


def kernel(v, adj, W0, W_out, b_out):
    return pl.pallas_call(
        _gcn_kernel,
        grid=(N // BLK,),
        in_specs=[
            pl.BlockSpec((N, FEATS), lambda i: (0, 0)),
            pl.BlockSpec((BLK, N), lambda i: (i, 0)),
            pl.BlockSpec((FEATS, HID), lambda i: (0, 0)),
            pl.BlockSpec((LABEL, N), lambda i: (0, 0)),
            pl.BlockSpec((LABEL,), lambda i: (0,)),
        ],
        out_specs=pl.BlockSpec((LABEL,), lambda i: (0,)),
        out_shape=jax.ShapeDtypeStruct((LABEL,), jnp.float32),
        scratch_shapes=[pltpu.VMEM((N, HID), jnp.float32)],
    )(v, adj, W0, W_out, b_out)


# speedup vs baseline: 1.0669x; 1.0541x over previous
import jax
import jax.numpy as jnp
from jax.experimental import pallas as pl
from jax.experimental.pallas import tpu as pltpu

N = 4096
FEATS = 128
HID = 64
LABEL = 10
BLK = 512


def _gcn_kernel(v_ref, adj_ref, w0_ref, wout_ref, bout_ref, out_ref,
                support_ref):
    i = pl.program_id(0)

    @pl.when(i == 0)
    def _init():
        support_ref[:] = jax.lax.dot_general(
            v_ref[:], w0_ref[:], (((1,), (1,)), ((), ())),
            preferred_element_type=jnp.float32)
        out_ref[:] = bout_ref[:]

    h = jnp.dot(adj_ref[:], support_ref[:],
                preferred_element_type=jnp.float32)
    s = jnp.sum(jnp.maximum(h, 0.0), axis=1)[None, :]
    partial = jax.lax.dot_general(
        s, wout_ref[:, pl.ds(i * BLK, BLK)], (((1,), (1,)), ((), ())),
        preferred_element_type=jnp.float32)
    out_ref[:] += partial[0]


def kernel(v, adj, W0, W_out, b_out):
    return pl.pallas_call(
        _gcn_kernel,
        grid=(N // BLK,),
        in_specs=[
            pl.BlockSpec((N, FEATS), lambda i: (0, 0)),
            pl.BlockSpec((BLK, N), lambda i: (i, 0)),
            pl.BlockSpec((HID, FEATS), lambda i: (0, 0)),
            pl.BlockSpec((LABEL, N), lambda i: (0, 0)),
            pl.BlockSpec((LABEL,), lambda i: (0,)),
        ],
        out_specs=pl.BlockSpec((LABEL,), lambda i: (0,)),
        out_shape=jax.ShapeDtypeStruct((LABEL,), jnp.float32),
        scratch_shapes=[pltpu.VMEM((N, HID), jnp.float32)],
    )(v, adj, W0.T, W_out, b_out)
